# baseline (speedup 1.0000x reference)
import functools

import jax
import jax.numpy as jnp
from jax import lax
from jax.experimental import pallas as pl
from jax.experimental.pallas import tpu as pltpu
from jax.experimental.pallas import tpu_sc as plsc

NUM_TOKENS = 1000000
DIM = 64
BATCH = 4096
SEQ = 200

NC = 2
NS = 16
NW = NC * NS

TOTAL = BATCH * SEQ
ROWS_PER_W = TOTAL // NW
IPG = 128
IR_PER_W = ROWS_PER_W // IPG
GPC = 4
CHUNK = GPC * IPG
NCHUNK = ROWS_PER_W // CHUNK

_mesh = plsc.VectorSubcoreMesh(core_axis_name="c", subcore_axis_name="s")


@functools.partial(
    pl.kernel,
    out_type=jax.ShapeDtypeStruct((TOTAL, DIM), jnp.float32),
    mesh=_mesh,
    scratch_types=[
        pltpu.VMEM((IR_PER_W, IPG), jnp.int32),
        pltpu.VMEM((CHUNK, DIM), jnp.float32),
        pltpu.SemaphoreType.DMA,
    ],
    compiler_params=pltpu.CompilerParams(use_tc_tiling_on_sc=False),
)
def _emb_lookup(table_hbm, idx_hbm, out_hbm, idx_v, buf, gsem):
    wid = lax.axis_index("s") * NC + lax.axis_index("c")
    irow0 = wid * IR_PER_W
    pltpu.sync_copy(idx_hbm.at[pl.ds(irow0, IR_PER_W)], idx_v)

    def chunk_body(g, carry):
        row0 = wid * ROWS_PER_W + g * CHUNK
        copies = [
            pltpu.async_copy(
                table_hbm.at[idx_v.at[g * GPC + j]],
                buf.at[pl.ds(j * IPG, IPG)],
                gsem,
            )
            for j in range(GPC)
        ]
        for c in copies:
            c.wait()
        pltpu.sync_copy(buf, out_hbm.at[pl.ds(row0, CHUNK)])
        return carry

    lax.fori_loop(0, NCHUNK, chunk_body, 0)


def kernel(x, emb_weight):
    idx = x.reshape(TOTAL // IPG, IPG)
    out = _emb_lookup(emb_weight, idx)
    return out.reshape(BATCH, SEQ, DIM)


# speedup vs baseline: 1.0249x; 1.0249x over previous
import functools

import jax
import jax.numpy as jnp
from jax import lax
from jax.experimental import pallas as pl
from jax.experimental.pallas import tpu as pltpu
from jax.experimental.pallas import tpu_sc as plsc

NUM_TOKENS = 1000000
DIM = 64
BATCH = 4096
SEQ = 200

NC = 2
NS = 16
NW = NC * NS

TOTAL = BATCH * SEQ
ROWS_PER_W = TOTAL // NW
IPG = 128
IR_PER_W = ROWS_PER_W // IPG
GPC = 4
CHUNK = GPC * IPG
NCHUNK = ROWS_PER_W // CHUNK

_mesh = plsc.VectorSubcoreMesh(core_axis_name="c", subcore_axis_name="s")


@functools.partial(
    pl.kernel,
    out_type=jax.ShapeDtypeStruct((TOTAL, DIM), jnp.float32),
    mesh=_mesh,
    scratch_types=[
        pltpu.VMEM((IR_PER_W, IPG), jnp.int32),
        pltpu.VMEM((CHUNK, DIM), jnp.float32),
        pltpu.VMEM((CHUNK, DIM), jnp.float32),
        pltpu.SemaphoreType.DMA,
        pltpu.SemaphoreType.DMA,
        pltpu.SemaphoreType.DMA,
        pltpu.SemaphoreType.DMA,
    ],
    compiler_params=pltpu.CompilerParams(use_tc_tiling_on_sc=False),
)
def _emb_lookup(table_hbm, idx_hbm, out_hbm, idx_v, buf0, buf1,
                gsem0, gsem1, osem0, osem1):
    wid = lax.axis_index("s") * NC + lax.axis_index("c")
    irow0 = wid * IR_PER_W
    row_base = wid * ROWS_PER_W
    pltpu.sync_copy(idx_hbm.at[pl.ds(irow0, IR_PER_W)], idx_v)

    def fire_gathers(g, buf, gsem):
        for j in range(GPC):
            pltpu.async_copy(
                table_hbm.at[idx_v.at[g * GPC + j]],
                buf.at[pl.ds(j * IPG, IPG)],
                gsem,
            )

    def wait_gathers(buf, gsem):
        for j in range(GPC):
            pltpu.make_async_copy(
                table_hbm.at[idx_v.at[j]],
                buf.at[pl.ds(j * IPG, IPG)],
                gsem,
            ).wait()

    def fire_out(g, buf, osem):
        pltpu.async_copy(buf, out_hbm.at[pl.ds(row_base + g * CHUNK, CHUNK)],
                         osem)

    def wait_out(buf, osem):
        pltpu.make_async_copy(
            buf, out_hbm.at[pl.ds(row_base, CHUNK)], osem
        ).wait()

    fire_gathers(0, buf0, gsem0)

    def pair_body(t, carry):
        g0 = 2 * t
        @pl.when(t > 0)
        def _():
            wait_out(buf1, osem1)

        fire_gathers(g0 + 1, buf1, gsem1)
        wait_gathers(buf0, gsem0)
        fire_out(g0, buf0, osem0)

        wait_out(buf0, osem0)

        @pl.when(g0 + 2 < NCHUNK)
        def _():
            fire_gathers(g0 + 2, buf0, gsem0)

        wait_gathers(buf1, gsem1)
        fire_out(g0 + 1, buf1, osem1)
        return carry

    lax.fori_loop(0, NCHUNK // 2, pair_body, 0)
    wait_out(buf1, osem1)


def kernel(x, emb_weight):
    idx = x.reshape(TOTAL // IPG, IPG)
    out = _emb_lookup(emb_weight, idx)
    return out.reshape(BATCH, SEQ, DIM)
